# baseline (speedup 1.0000x reference)
import functools

import jax
import jax.numpy as jnp
from jax import lax
from jax.experimental import pallas as pl
from jax.experimental.pallas import tpu as pltpu
from jax.experimental.pallas import tpu_sc as plsc

N = 10000
M = 10000
E = 320000
D = 128
NEG_SLOPE = 0.2
SCALE = 1.0 / (D ** 0.5)

NC = 2
NS = 16
NW = NC * NS
L = 16

EP = E // NW
CHUNK = 80
NCHUNKS = EP // CHUNK
GP = CHUNK // L
SEG_PAD = 10240
SEG_TILE = SEG_PAD // NW
ROWS_TILE = SEG_PAD // NS
NEG_INIT = -1e30


@functools.cache
def _mesh():
    return plsc.VectorSubcoreMesh(
        core_axis_name="c", subcore_axis_name="s",
        num_cores=NC, num_subcores=NS)


def _fill(ref, n, value, dtype):
    def body(i, _):
        ref[pl.ds(i * L, L)] = jnp.full((L,), value, dtype)
        return 0
    lax.fori_loop(0, n // L, body, 0)


def _scatter_max(ref, idx, val):
    def cond(pending):
        return jnp.any(pending)

    def body(pending):
        cur = plsc.load_gather(ref, [idx])
        plsc.store_scatter(ref, [idx], jnp.maximum(cur, val), mask=pending)
        cur2 = plsc.load_gather(ref, [idx])
        return pending & (cur2 < val)

    init = val > plsc.load_gather(ref, [idx])
    lax.while_loop(cond, body, init)


def _edge_scores_body(tab_a, idx_a, tab_b, idx_b, key_idx, e_out, pmax_out,
                      ia_v, ib_v, ik_v, e_v, mx_v, ar0_v, br0_v, ar1_v, br1_v,
                      eb_v, sa0, sb0, sa1, sb1):
    wid = lax.axis_index("c") * NS + lax.axis_index("s")
    base = pl.multiple_of(wid * EP, 8)
    pltpu.sync_copy(idx_a.at[pl.ds(base, EP)], ia_v)
    pltpu.sync_copy(idx_b.at[pl.ds(base, EP)], ib_v)
    pltpu.sync_copy(key_idx.at[pl.ds(base, EP)], ik_v)
    _fill(mx_v, SEG_PAD, NEG_INIT, jnp.float32)
    iota = lax.iota(jnp.int32, L)
    bufs = ((ar0_v, br0_v, sa0, sb0), (ar1_v, br1_v, sa1, sb1))

    def issue(ch, b):
        off = ch * CHUNK
        ar_v, br_v, sem_a, sem_b = bufs[b]
        pltpu.async_copy(tab_a.at[ia_v.at[pl.ds(off, CHUNK)]], ar_v, sem_a)
        pltpu.async_copy(tab_b.at[ib_v.at[pl.ds(off, CHUNK)]], br_v, sem_b)

    def wait(ch, b):
        off = ch * CHUNK
        ar_v, br_v, sem_a, sem_b = bufs[b]
        pltpu.make_async_copy(tab_a.at[ia_v.at[pl.ds(off, CHUNK)]], ar_v,
                              sem_a).wait()
        pltpu.make_async_copy(tab_b.at[ib_v.at[pl.ds(off, CHUNK)]], br_v,
                              sem_b).wait()

    def compute(ch, b):
        off = ch * CHUNK
        ar_v, br_v, _, _ = bufs[b]

        def group_body(g, _):
            def row_body(i, _):
                r = g * L + i
                acc = ar_v[r, pl.ds(0, L)] * br_v[r, pl.ds(0, L)]
                for k in range(1, D // L):
                    sl = pl.ds(k * L, L)
                    acc = acc + ar_v[r, sl] * br_v[r, sl]
                eb_v[pl.ds(i * L, L)] = acc
                return 0

            lax.fori_loop(0, L, row_body, 0)
            acc = plsc.load_gather(eb_v, [iota * L])
            for c in range(1, L):
                acc = acc + plsc.load_gather(eb_v, [iota * L + c])
            e16 = jnp.where(acc >= 0, acc, NEG_SLOPE * acc)
            e_v[pl.ds(off + g * L, L)] = e16
            k16 = ik_v[pl.ds(off + g * L, L)]
            _scatter_max(mx_v, k16, e16)
            return 0

        lax.fori_loop(0, GP, group_body, 0)

    issue(0, 0)
    issue(1, 1)

    def pair_body(p, _):
        for b in range(2):
            ch = p * 2 + b
            wait(ch, b)
            compute(ch, b)
            nxt = ch + 2

            @pl.when(nxt < NCHUNKS)
            def _():
                issue(nxt, b)

        return 0

    lax.fori_loop(0, NCHUNKS // 2, pair_body, 0)
    if NCHUNKS % 2:
        wait(NCHUNKS - 1, 0)
        compute(NCHUNKS - 1, 0)
    pltpu.sync_copy(e_v, e_out.at[pl.ds(base, EP)])
    pltpu.sync_copy(mx_v, pmax_out.at[pl.ds(pl.multiple_of(wid * SEG_PAD, 8),
                                            SEG_PAD)])


@functools.cache
def _edge_scores():
    return pl.kernel(
        _edge_scores_body,
        out_type=[jax.ShapeDtypeStruct((E,), jnp.float32),
                  jax.ShapeDtypeStruct((NW * SEG_PAD,), jnp.float32)],
        mesh=_mesh(),
        compiler_params=pltpu.CompilerParams(needs_layout_passes=False),
        scratch_types=[
            pltpu.VMEM((EP,), jnp.int32),
            pltpu.VMEM((EP,), jnp.int32),
            pltpu.VMEM((EP,), jnp.int32),
            pltpu.VMEM((EP,), jnp.float32),
            pltpu.VMEM((SEG_PAD,), jnp.float32),
            pltpu.VMEM((CHUNK, D), jnp.float32),
            pltpu.VMEM((CHUNK, D), jnp.float32),
            pltpu.VMEM((CHUNK, D), jnp.float32),
            pltpu.VMEM((CHUNK, D), jnp.float32),
            pltpu.VMEM((L * L,), jnp.float32),
            pltpu.SemaphoreType.DMA,
            pltpu.SemaphoreType.DMA,
            pltpu.SemaphoreType.DMA,
            pltpu.SemaphoreType.DMA,
        ],
    )


def _make_combine(fold, finalize):
    def body(src_hbm, dst_out, acc_v, tmp_v):
        wid = lax.axis_index("c") * NS + lax.axis_index("s")
        off = pl.multiple_of(wid * SEG_TILE, 8)
        pltpu.sync_copy(src_hbm.at[pl.ds(off, SEG_TILE)], acc_v)

        def body_k(k, _):
            src_off = pl.multiple_of(k * SEG_PAD + off, 8)
            pltpu.sync_copy(src_hbm.at[pl.ds(src_off, SEG_TILE)], tmp_v)

            def vb(i, _):
                sl = pl.ds(i * L, L)
                acc_v[sl] = fold(acc_v[sl], tmp_v[sl])
                return 0

            lax.fori_loop(0, SEG_TILE // L, vb, 0)
            return 0

        lax.fori_loop(1, NW, body_k, 0)
        if finalize is not None:
            def vf(i, _):
                sl = pl.ds(i * L, L)
                acc_v[sl] = finalize(acc_v[sl])
                return 0
            lax.fori_loop(0, SEG_TILE // L, vf, 0)
        pltpu.sync_copy(acc_v, dst_out.at[pl.ds(off, SEG_TILE)])

    return pl.kernel(
        body,
        out_type=jax.ShapeDtypeStruct((SEG_PAD,), jnp.float32),
        mesh=_mesh(),
        compiler_params=pltpu.CompilerParams(needs_layout_passes=False),
        scratch_types=[
            pltpu.VMEM((SEG_TILE,), jnp.float32),
            pltpu.VMEM((SEG_TILE,), jnp.float32),
        ],
    )


@functools.cache
def _combine_max():
    return _make_combine(jnp.maximum, None)


@functools.cache
def _combine_sum_recip():
    return _make_combine(lambda a, b: a + b, lambda a: 1.0 / (a + 1e-16))


def _weights_body(key_idx, e_hbm, m_hbm, w_out, psum_out,
                  ik_v, e_v, w_v, m_v, ps_v):
    wid = lax.axis_index("c") * NS + lax.axis_index("s")
    base = pl.multiple_of(wid * EP, 8)
    pltpu.sync_copy(key_idx.at[pl.ds(base, EP)], ik_v)
    pltpu.sync_copy(e_hbm.at[pl.ds(base, EP)], e_v)
    pltpu.sync_copy(m_hbm, m_v)
    _fill(ps_v, SEG_PAD, 0.0, jnp.float32)

    def group_body(g, _):
        sl = pl.ds(g * L, L)
        k16 = ik_v[sl]
        e16 = e_v[sl]
        m16 = plsc.load_gather(m_v, [k16])
        w16 = jnp.exp(e16 - m16)
        w_v[sl] = w16
        plsc.addupdate_scatter(ps_v, [k16], w16)
        return 0

    lax.fori_loop(0, EP // L, group_body, 0)
    pltpu.sync_copy(w_v, w_out.at[pl.ds(base, EP)])
    pltpu.sync_copy(ps_v, psum_out.at[pl.ds(pl.multiple_of(wid * SEG_PAD, 8),
                                            SEG_PAD)])


@functools.cache
def _weights():
    return pl.kernel(
        _weights_body,
        out_type=[jax.ShapeDtypeStruct((E,), jnp.float32),
                  jax.ShapeDtypeStruct((NW * SEG_PAD,), jnp.float32)],
        mesh=_mesh(),
        compiler_params=pltpu.CompilerParams(needs_layout_passes=False),
        scratch_types=[
            pltpu.VMEM((EP,), jnp.int32),
            pltpu.VMEM((EP,), jnp.float32),
            pltpu.VMEM((EP,), jnp.float32),
            pltpu.VMEM((SEG_PAD,), jnp.float32),
            pltpu.VMEM((SEG_PAD,), jnp.float32),
        ],
    )


def _messages_body(msg_tab, other_idx, key_idx, w_hbm, rinv_hbm, acc_out,
                   io_v, rinv_v, rows0_v, rows1_v, kch0_v, kch1_v, wch0_v,
                   wch1_v, wb_v, acc_sh, sr0, sk0, sw0, sr1, sk1, sw1):
    cid = lax.axis_index("c")
    sid = lax.axis_index("s")
    wid = cid * NS + sid
    base = pl.multiple_of(wid * EP, 8)
    pltpu.sync_copy(other_idx.at[pl.ds(base, EP)], io_v)
    pltpu.sync_copy(rinv_hbm, rinv_v)
    bufs = ((rows0_v, kch0_v, wch0_v, sr0, sk0, sw0),
            (rows1_v, kch1_v, wch1_v, sr1, sk1, sw1))

    def zrow(r, _):
        def zcol(i, _):
            rows0_v[r, pl.ds(i * L, L)] = jnp.zeros((L,), jnp.float32)
            return 0
        lax.fori_loop(0, D // L, zcol, 0)
        return 0
    lax.fori_loop(0, CHUNK, zrow, 0)
    rbase = sid * ROWS_TILE
    for j in range(ROWS_TILE // CHUNK):
        pltpu.sync_copy(rows0_v, acc_sh.at[pl.ds(rbase + j * CHUNK, CHUNK)])

    def issue(ch, b):
        off = ch * CHUNK
        goff = pl.multiple_of(base + off, 8)
        rows_v, kch_v, wch_v, sr, sk, sw = bufs[b]
        pltpu.async_copy(msg_tab.at[io_v.at[pl.ds(off, CHUNK)]], rows_v, sr)
        pltpu.async_copy(key_idx.at[pl.ds(goff, CHUNK)], kch_v, sk)
        pltpu.async_copy(w_hbm.at[pl.ds(goff, CHUNK)], wch_v, sw)

    def wait(ch, b):
        off = ch * CHUNK
        goff = pl.multiple_of(base + off, 8)
        rows_v, kch_v, wch_v, sr, sk, sw = bufs[b]
        pltpu.make_async_copy(msg_tab.at[io_v.at[pl.ds(off, CHUNK)]], rows_v,
                              sr).wait()
        pltpu.make_async_copy(key_idx.at[pl.ds(goff, CHUNK)], kch_v,
                              sk).wait()
        pltpu.make_async_copy(w_hbm.at[pl.ds(goff, CHUNK)], wch_v, sw).wait()

    def compute(ch, b):
        rows_v, kch_v, wch_v, _, _, _ = bufs[b]

        def group_body(g, _):
            sl = pl.ds(g * L, L)
            k16 = kch_v[sl]
            w16 = wch_v[sl]
            r16 = plsc.load_gather(rinv_v, [k16])
            wb_v[pl.ds(0, L)] = w16 * r16 * SCALE

            def row_body(i, _):
                r = g * L + i
                wb = plsc.load_gather(wb_v, [jnp.full((L,), i, jnp.int32)])
                for k in range(D // L):
                    rsl = pl.ds(k * L, L)
                    rows_v[r, rsl] = rows_v[r, rsl] * wb
                return 0

            lax.fori_loop(0, L, row_body, 0)
            return 0

        lax.fori_loop(0, GP, group_body, 0)
        pltpu.sync_copy(rows_v, acc_sh.at[kch_v], add=True)

    issue(0, 0)
    issue(1, 1)
    plsc.subcore_barrier()

    def pair_body(p, _):
        for b in range(2):
            ch = p * 2 + b
            wait(ch, b)
            compute(ch, b)
            nxt = ch + 2

            @pl.when(nxt < NCHUNKS)
            def _():
                issue(nxt, b)

        return 0

    lax.fori_loop(0, NCHUNKS // 2, pair_body, 0)
    if NCHUNKS % 2:
        wait(NCHUNKS - 1, 0)
        compute(NCHUNKS - 1, 0)
    plsc.subcore_barrier()
    for j in range(ROWS_TILE // CHUNK):
        r0 = rbase + j * CHUNK
        pltpu.sync_copy(acc_sh.at[pl.ds(r0, CHUNK)],
                        acc_out.at[cid, pl.ds(r0, CHUNK)])


@functools.cache
def _messages():
    return pl.kernel(
        _messages_body,
        out_type=jax.ShapeDtypeStruct((NC, SEG_PAD, D), jnp.float32),
        mesh=_mesh(),
        compiler_params=pltpu.CompilerParams(needs_layout_passes=False),
        scratch_types=[
            pltpu.VMEM((EP,), jnp.int32),
            pltpu.VMEM((SEG_PAD,), jnp.float32),
            pltpu.VMEM((CHUNK, D), jnp.float32),
            pltpu.VMEM((CHUNK, D), jnp.float32),
            pltpu.VMEM((CHUNK,), jnp.int32),
            pltpu.VMEM((CHUNK,), jnp.int32),
            pltpu.VMEM((CHUNK,), jnp.float32),
            pltpu.VMEM((CHUNK,), jnp.float32),
            pltpu.VMEM((L,), jnp.float32),
            pltpu.VMEM_SHARED((SEG_PAD, D), jnp.float32),
            pltpu.SemaphoreType.DMA,
            pltpu.SemaphoreType.DMA,
            pltpu.SemaphoreType.DMA,
            pltpu.SemaphoreType.DMA,
            pltpu.SemaphoreType.DMA,
            pltpu.SemaphoreType.DMA,
        ],
    )


_MMB = 1000


def _mm_body3(x_ref, w1_ref, w2_ref, w3_ref, o1, o2, o3):
    xv = x_ref[...]
    o1[...] = jnp.dot(xv, w1_ref[...], preferred_element_type=jnp.float32)
    o2[...] = jnp.dot(xv, w2_ref[...], preferred_element_type=jnp.float32)
    o3[...] = jnp.dot(xv, w3_ref[...], preferred_element_type=jnp.float32)


def _mm3(x, w1, w2, w3):
    wspec = pl.BlockSpec((D, D), lambda i: (0, 0))
    rspec = pl.BlockSpec((_MMB, D), lambda i: (i, 0))
    return pl.pallas_call(
        _mm_body3,
        grid=(x.shape[0] // _MMB,),
        in_specs=[rspec, wspec, wspec, wspec],
        out_specs=[rspec, rspec, rspec],
        out_shape=[jax.ShapeDtypeStruct((x.shape[0], D), jnp.float32)] * 3,
    )(x, w1, w2, w3)


def _mm_body1(x_ref, w_ref, o_ref):
    o_ref[...] = jnp.dot(x_ref[...], w_ref[...],
                         preferred_element_type=jnp.float32)


def _mm1(x, w):
    wspec = pl.BlockSpec((D, D), lambda i: (0, 0))
    rspec = pl.BlockSpec((_MMB, D), lambda i: (i, 0))
    return pl.pallas_call(
        _mm_body1,
        grid=(x.shape[0] // _MMB,),
        in_specs=[rspec, wspec],
        out_specs=rspec,
        out_shape=jax.ShapeDtypeStruct((x.shape[0], D), jnp.float32),
    )(x, w)


_CB = 1024


def _comb_mm_body(acc_ref, w4_ref, w5_ref, o1, o2):
    pr = acc_ref[0] + acc_ref[1]
    o1[...] = jnp.dot(pr, w4_ref[...], preferred_element_type=jnp.float32)
    o2[...] = jnp.dot(pr, w5_ref[...], preferred_element_type=jnp.float32)


def _comb_mm(acc, w4, w5):
    return pl.pallas_call(
        _comb_mm_body,
        grid=(SEG_PAD // _CB,),
        in_specs=[pl.BlockSpec((NC, _CB, D), lambda i: (0, i, 0)),
                  pl.BlockSpec((D, D), lambda i: (0, 0)),
                  pl.BlockSpec((D, D), lambda i: (0, 0))],
        out_specs=[pl.BlockSpec((_CB, D), lambda i: (i, 0)),
                   pl.BlockSpec((_CB, D), lambda i: (i, 0))],
        out_shape=[jax.ShapeDtypeStruct((SEG_PAD, D), jnp.float32)] * 2,
    )(acc, w4, w5)


def _comb_out_body(acc_ref, o_ref):
    o_ref[...] = acc_ref[0] + acc_ref[1]


def _comb_out(acc):
    return pl.pallas_call(
        _comb_out_body,
        grid=(SEG_PAD // _CB,),
        in_specs=[pl.BlockSpec((NC, _CB, D), lambda i: (0, i, 0))],
        out_specs=pl.BlockSpec((_CB, D), lambda i: (i, 0)),
        out_shape=jax.ShapeDtypeStruct((SEG_PAD, D), jnp.float32),
    )(acc)


def kernel(p, q, hedge_index, W1, W2, W3, W4, W5, W6):
    node_idx = hedge_index[0]
    hedge_idx = hedge_index[1]
    q_p, q_pp, p_ppp = _mm3(q, W1, W2, W6)
    q_ppp = _mm1(p, W3)
    e1, pmax1 = _edge_scores()(q_ppp, node_idx, q_pp, hedge_idx, node_idx)
    m1 = _combine_max()(pmax1)
    w1, ps1 = _weights()(node_idx, e1, m1)
    rinv1 = _combine_sum_recip()(ps1)
    acc1 = _messages()(q_p, hedge_idx, node_idx, w1, rinv1)
    p_p, p_pp = _comb_mm(acc1, W4, W5)
    e2, pmax2 = _edge_scores()(p_pp, node_idx, p_ppp, hedge_idx, hedge_idx)
    m2 = _combine_max()(pmax2)
    w2, ps2 = _weights()(hedge_idx, e2, m2)
    rinv2 = _combine_sum_recip()(ps2)
    acc2 = _messages()(p_p, node_idx, hedge_idx, w2, rinv2)
    return _comb_out(acc2)[:M]


# speedup vs baseline: 1.0253x; 1.0253x over previous
import functools

import jax
import jax.numpy as jnp
from jax import lax
from jax.experimental import pallas as pl
from jax.experimental.pallas import tpu as pltpu
from jax.experimental.pallas import tpu_sc as plsc

N = 10000
M = 10000
E = 320000
D = 128
NEG_SLOPE = 0.2
SCALE = 1.0 / (D ** 0.5)

NC = 2
NS = 16
NW = NC * NS
L = 16

EP = E // NW
CHUNK = 80
NCHUNKS = EP // CHUNK
GP = CHUNK // L
SEG_PAD = 10240
SEG_TILE = SEG_PAD // NW
ROWS_TILE = SEG_PAD // NS
NEG_INIT = -1e30


@functools.cache
def _mesh():
    return plsc.VectorSubcoreMesh(
        core_axis_name="c", subcore_axis_name="s",
        num_cores=NC, num_subcores=NS)


def _fill(ref, n, value, dtype):
    def body(i, _):
        ref[pl.ds(i * L, L)] = jnp.full((L,), value, dtype)
        return 0
    lax.fori_loop(0, n // L, body, 0)


def _scatter_max(ref, idx, val):
    def cond(pending):
        return jnp.any(pending)

    def body(pending):
        cur = plsc.load_gather(ref, [idx])
        plsc.store_scatter(ref, [idx], jnp.maximum(cur, val), mask=pending)
        cur2 = plsc.load_gather(ref, [idx])
        return pending & (cur2 < val)

    init = val > plsc.load_gather(ref, [idx])
    lax.while_loop(cond, body, init)


_NBUF = 4


def _make_edge_scores_body(key_is_a):
    def body(tab_a, idx_a, tab_b, idx_b, e_out, pmax_out,
             ia_v, ib_v, e_v, mx_v,
             ar0_v, br0_v, ar1_v, br1_v, ar2_v, br2_v, ar3_v, br3_v,
             eb_v, sa0, sb0, sa1, sb1, sa2, sb2, sa3, sb3):
        wid = lax.axis_index("c") * NS + lax.axis_index("s")
        base = pl.multiple_of(wid * EP, 8)
        pltpu.sync_copy(idx_a.at[pl.ds(base, EP)], ia_v)
        pltpu.sync_copy(idx_b.at[pl.ds(base, EP)], ib_v)
        ik_v = ia_v if key_is_a else ib_v
        _fill(mx_v, SEG_PAD, NEG_INIT, jnp.float32)
        iota = lax.iota(jnp.int32, L)
        bufs = ((ar0_v, br0_v, sa0, sb0), (ar1_v, br1_v, sa1, sb1),
                (ar2_v, br2_v, sa2, sb2), (ar3_v, br3_v, sa3, sb3))

        def issue(ch, b):
            off = ch * CHUNK
            ar_v, br_v, sem_a, sem_b = bufs[b]
            pltpu.async_copy(tab_a.at[ia_v.at[pl.ds(off, CHUNK)]], ar_v, sem_a)
            pltpu.async_copy(tab_b.at[ib_v.at[pl.ds(off, CHUNK)]], br_v, sem_b)

        def wait(ch, b):
            off = ch * CHUNK
            ar_v, br_v, sem_a, sem_b = bufs[b]
            pltpu.make_async_copy(tab_a.at[ia_v.at[pl.ds(off, CHUNK)]], ar_v,
                                  sem_a).wait()
            pltpu.make_async_copy(tab_b.at[ib_v.at[pl.ds(off, CHUNK)]], br_v,
                                  sem_b).wait()

        def compute(ch, b):
            off = ch * CHUNK
            ar_v, br_v, _, _ = bufs[b]

            def group_body(g, _):
                def row_body(i, _):
                    r = g * L + i
                    acc = ar_v[r, pl.ds(0, L)] * br_v[r, pl.ds(0, L)]
                    for k in range(1, D // L):
                        sl = pl.ds(k * L, L)
                        acc = acc + ar_v[r, sl] * br_v[r, sl]
                    eb_v[pl.ds(i * L, L)] = acc
                    return 0

                lax.fori_loop(0, L, row_body, 0, unroll=True)
                acc = plsc.load_gather(eb_v, [iota * L])
                for c in range(1, L):
                    acc = acc + plsc.load_gather(eb_v, [iota * L + c])
                e16 = jnp.where(acc >= 0, acc, NEG_SLOPE * acc)
                e_v[pl.ds(off + g * L, L)] = e16
                k16 = ik_v[pl.ds(off + g * L, L)]
                _scatter_max(mx_v, k16, e16)
                return 0

            lax.fori_loop(0, GP, group_body, 0)

        for b in range(_NBUF):
            issue(b, b)

        def ring_body(p, _):
            for b in range(_NBUF):
                ch = p * _NBUF + b
                wait(ch, b)
                compute(ch, b)
                nxt = ch + _NBUF

                @pl.when(nxt < NCHUNKS)
                def _():
                    issue(nxt, b)

            return 0

        lax.fori_loop(0, NCHUNKS // _NBUF, ring_body, 0)
        for b in range(NCHUNKS % _NBUF):
            ch = (NCHUNKS // _NBUF) * _NBUF + b
            wait(ch, b)
            compute(ch, b)
        pltpu.sync_copy(e_v, e_out.at[pl.ds(base, EP)])
        pltpu.sync_copy(mx_v,
                        pmax_out.at[pl.ds(pl.multiple_of(wid * SEG_PAD, 8),
                                          SEG_PAD)])

    return body


@functools.cache
def _edge_scores(key_is_a):
    return pl.kernel(
        _make_edge_scores_body(key_is_a),
        out_type=[jax.ShapeDtypeStruct((E,), jnp.float32),
                  jax.ShapeDtypeStruct((NW * SEG_PAD,), jnp.float32)],
        mesh=_mesh(),
        compiler_params=pltpu.CompilerParams(needs_layout_passes=False),
        scratch_types=[
            pltpu.VMEM((EP,), jnp.int32),
            pltpu.VMEM((EP,), jnp.int32),
            pltpu.VMEM((EP,), jnp.float32),
            pltpu.VMEM((SEG_PAD,), jnp.float32),
        ] + [pltpu.VMEM((CHUNK, D), jnp.float32)] * (2 * _NBUF) + [
            pltpu.VMEM((L * L,), jnp.float32),
        ] + [pltpu.SemaphoreType.DMA] * (2 * _NBUF),
    )


def _make_combine(fold, finalize):
    def body(src_hbm, dst_out, acc_v, tmp_v):
        wid = lax.axis_index("c") * NS + lax.axis_index("s")
        off = pl.multiple_of(wid * SEG_TILE, 8)
        pltpu.sync_copy(src_hbm.at[pl.ds(off, SEG_TILE)], acc_v)

        def body_k(k, _):
            src_off = pl.multiple_of(k * SEG_PAD + off, 8)
            pltpu.sync_copy(src_hbm.at[pl.ds(src_off, SEG_TILE)], tmp_v)

            def vb(i, _):
                sl = pl.ds(i * L, L)
                acc_v[sl] = fold(acc_v[sl], tmp_v[sl])
                return 0

            lax.fori_loop(0, SEG_TILE // L, vb, 0)
            return 0

        lax.fori_loop(1, NW, body_k, 0)
        if finalize is not None:
            def vf(i, _):
                sl = pl.ds(i * L, L)
                acc_v[sl] = finalize(acc_v[sl])
                return 0
            lax.fori_loop(0, SEG_TILE // L, vf, 0)
        pltpu.sync_copy(acc_v, dst_out.at[pl.ds(off, SEG_TILE)])

    return pl.kernel(
        body,
        out_type=jax.ShapeDtypeStruct((SEG_PAD,), jnp.float32),
        mesh=_mesh(),
        compiler_params=pltpu.CompilerParams(needs_layout_passes=False),
        scratch_types=[
            pltpu.VMEM((SEG_TILE,), jnp.float32),
            pltpu.VMEM((SEG_TILE,), jnp.float32),
        ],
    )


@functools.cache
def _combine_max():
    return _make_combine(jnp.maximum, None)


@functools.cache
def _combine_sum_recip():
    return _make_combine(lambda a, b: a + b, lambda a: 1.0 / (a + 1e-16))


def _weights_body(key_idx, e_hbm, m_hbm, w_out, psum_out,
                  ik_v, e_v, w_v, m_v, ps_v):
    wid = lax.axis_index("c") * NS + lax.axis_index("s")
    base = pl.multiple_of(wid * EP, 8)
    pltpu.sync_copy(key_idx.at[pl.ds(base, EP)], ik_v)
    pltpu.sync_copy(e_hbm.at[pl.ds(base, EP)], e_v)
    pltpu.sync_copy(m_hbm, m_v)
    _fill(ps_v, SEG_PAD, 0.0, jnp.float32)

    def group_body(g, _):
        sl = pl.ds(g * L, L)
        k16 = ik_v[sl]
        e16 = e_v[sl]
        m16 = plsc.load_gather(m_v, [k16])
        w16 = jnp.exp(e16 - m16)
        w_v[sl] = w16
        plsc.addupdate_scatter(ps_v, [k16], w16)
        return 0

    lax.fori_loop(0, EP // L, group_body, 0)
    pltpu.sync_copy(w_v, w_out.at[pl.ds(base, EP)])
    pltpu.sync_copy(ps_v, psum_out.at[pl.ds(pl.multiple_of(wid * SEG_PAD, 8),
                                            SEG_PAD)])


@functools.cache
def _weights():
    return pl.kernel(
        _weights_body,
        out_type=[jax.ShapeDtypeStruct((E,), jnp.float32),
                  jax.ShapeDtypeStruct((NW * SEG_PAD,), jnp.float32)],
        mesh=_mesh(),
        compiler_params=pltpu.CompilerParams(needs_layout_passes=False),
        scratch_types=[
            pltpu.VMEM((EP,), jnp.int32),
            pltpu.VMEM((EP,), jnp.float32),
            pltpu.VMEM((EP,), jnp.float32),
            pltpu.VMEM((SEG_PAD,), jnp.float32),
            pltpu.VMEM((SEG_PAD,), jnp.float32),
        ],
    )


def _messages_body(msg_tab, other_idx, key_idx, w_hbm, rinv_hbm, acc_out,
                   io_v, rinv_v, rows0_v, rows1_v, kch0_v, kch1_v, wch0_v,
                   wch1_v, wb_v, acc_sh, sr0, sk0, sw0, sr1, sk1, sw1):
    cid = lax.axis_index("c")
    sid = lax.axis_index("s")
    wid = cid * NS + sid
    base = pl.multiple_of(wid * EP, 8)
    pltpu.sync_copy(other_idx.at[pl.ds(base, EP)], io_v)
    pltpu.sync_copy(rinv_hbm, rinv_v)
    bufs = ((rows0_v, kch0_v, wch0_v, sr0, sk0, sw0),
            (rows1_v, kch1_v, wch1_v, sr1, sk1, sw1))

    def zrow(r, _):
        def zcol(i, _):
            rows0_v[r, pl.ds(i * L, L)] = jnp.zeros((L,), jnp.float32)
            return 0
        lax.fori_loop(0, D // L, zcol, 0)
        return 0
    lax.fori_loop(0, CHUNK, zrow, 0)
    rbase = sid * ROWS_TILE
    for j in range(ROWS_TILE // CHUNK):
        pltpu.sync_copy(rows0_v, acc_sh.at[pl.ds(rbase + j * CHUNK, CHUNK)])

    def issue(ch, b):
        off = ch * CHUNK
        goff = pl.multiple_of(base + off, 8)
        rows_v, kch_v, wch_v, sr, sk, sw = bufs[b]
        pltpu.async_copy(msg_tab.at[io_v.at[pl.ds(off, CHUNK)]], rows_v, sr)
        pltpu.async_copy(key_idx.at[pl.ds(goff, CHUNK)], kch_v, sk)
        pltpu.async_copy(w_hbm.at[pl.ds(goff, CHUNK)], wch_v, sw)

    def wait(ch, b):
        off = ch * CHUNK
        goff = pl.multiple_of(base + off, 8)
        rows_v, kch_v, wch_v, sr, sk, sw = bufs[b]
        pltpu.make_async_copy(msg_tab.at[io_v.at[pl.ds(off, CHUNK)]], rows_v,
                              sr).wait()
        pltpu.make_async_copy(key_idx.at[pl.ds(goff, CHUNK)], kch_v,
                              sk).wait()
        pltpu.make_async_copy(w_hbm.at[pl.ds(goff, CHUNK)], wch_v, sw).wait()

    def compute(ch, b):
        rows_v, kch_v, wch_v, _, _, _ = bufs[b]

        def group_body(g, _):
            sl = pl.ds(g * L, L)
            k16 = kch_v[sl]
            w16 = wch_v[sl]
            r16 = plsc.load_gather(rinv_v, [k16])
            wb_v[pl.ds(0, L)] = w16 * r16 * SCALE

            def row_body(i, _):
                r = g * L + i
                wb = plsc.load_gather(wb_v, [jnp.full((L,), i, jnp.int32)])
                for k in range(D // L):
                    rsl = pl.ds(k * L, L)
                    rows_v[r, rsl] = rows_v[r, rsl] * wb
                return 0

            lax.fori_loop(0, L, row_body, 0, unroll=True)
            return 0

        lax.fori_loop(0, GP, group_body, 0)
        pltpu.sync_copy(rows_v, acc_sh.at[kch_v], add=True)

    issue(0, 0)
    issue(1, 1)
    plsc.subcore_barrier()

    def pair_body(p, _):
        for b in range(2):
            ch = p * 2 + b
            wait(ch, b)
            compute(ch, b)
            nxt = ch + 2

            @pl.when(nxt < NCHUNKS)
            def _():
                issue(nxt, b)

        return 0

    lax.fori_loop(0, NCHUNKS // 2, pair_body, 0)
    if NCHUNKS % 2:
        wait(NCHUNKS - 1, 0)
        compute(NCHUNKS - 1, 0)
    plsc.subcore_barrier()
    for j in range(ROWS_TILE // CHUNK):
        r0 = rbase + j * CHUNK
        pltpu.sync_copy(acc_sh.at[pl.ds(r0, CHUNK)],
                        acc_out.at[cid, pl.ds(r0, CHUNK)])


@functools.cache
def _messages():
    return pl.kernel(
        _messages_body,
        out_type=jax.ShapeDtypeStruct((NC, SEG_PAD, D), jnp.float32),
        mesh=_mesh(),
        compiler_params=pltpu.CompilerParams(needs_layout_passes=False),
        scratch_types=[
            pltpu.VMEM((EP,), jnp.int32),
            pltpu.VMEM((SEG_PAD,), jnp.float32),
            pltpu.VMEM((CHUNK, D), jnp.float32),
            pltpu.VMEM((CHUNK, D), jnp.float32),
            pltpu.VMEM((CHUNK,), jnp.int32),
            pltpu.VMEM((CHUNK,), jnp.int32),
            pltpu.VMEM((CHUNK,), jnp.float32),
            pltpu.VMEM((CHUNK,), jnp.float32),
            pltpu.VMEM((L,), jnp.float32),
            pltpu.VMEM_SHARED((SEG_PAD, D), jnp.float32),
            pltpu.SemaphoreType.DMA,
            pltpu.SemaphoreType.DMA,
            pltpu.SemaphoreType.DMA,
            pltpu.SemaphoreType.DMA,
            pltpu.SemaphoreType.DMA,
            pltpu.SemaphoreType.DMA,
        ],
    )


_MMB = 1000


def _mm_body3(x_ref, w1_ref, w2_ref, w3_ref, o1, o2, o3):
    xv = x_ref[...]
    o1[...] = jnp.dot(xv, w1_ref[...], preferred_element_type=jnp.float32)
    o2[...] = jnp.dot(xv, w2_ref[...], preferred_element_type=jnp.float32)
    o3[...] = jnp.dot(xv, w3_ref[...], preferred_element_type=jnp.float32)


def _mm3(x, w1, w2, w3):
    wspec = pl.BlockSpec((D, D), lambda i: (0, 0))
    rspec = pl.BlockSpec((_MMB, D), lambda i: (i, 0))
    return pl.pallas_call(
        _mm_body3,
        grid=(x.shape[0] // _MMB,),
        in_specs=[rspec, wspec, wspec, wspec],
        out_specs=[rspec, rspec, rspec],
        out_shape=[jax.ShapeDtypeStruct((x.shape[0], D), jnp.float32)] * 3,
    )(x, w1, w2, w3)


def _mm_body1(x_ref, w_ref, o_ref):
    o_ref[...] = jnp.dot(x_ref[...], w_ref[...],
                         preferred_element_type=jnp.float32)


def _mm1(x, w):
    wspec = pl.BlockSpec((D, D), lambda i: (0, 0))
    rspec = pl.BlockSpec((_MMB, D), lambda i: (i, 0))
    return pl.pallas_call(
        _mm_body1,
        grid=(x.shape[0] // _MMB,),
        in_specs=[rspec, wspec],
        out_specs=rspec,
        out_shape=jax.ShapeDtypeStruct((x.shape[0], D), jnp.float32),
    )(x, w)


_CB = 1024


def _comb_mm_body(acc_ref, w4_ref, w5_ref, o1, o2):
    pr = acc_ref[0] + acc_ref[1]
    o1[...] = jnp.dot(pr, w4_ref[...], preferred_element_type=jnp.float32)
    o2[...] = jnp.dot(pr, w5_ref[...], preferred_element_type=jnp.float32)


def _comb_mm(acc, w4, w5):
    return pl.pallas_call(
        _comb_mm_body,
        grid=(SEG_PAD // _CB,),
        in_specs=[pl.BlockSpec((NC, _CB, D), lambda i: (0, i, 0)),
                  pl.BlockSpec((D, D), lambda i: (0, 0)),
                  pl.BlockSpec((D, D), lambda i: (0, 0))],
        out_specs=[pl.BlockSpec((_CB, D), lambda i: (i, 0)),
                   pl.BlockSpec((_CB, D), lambda i: (i, 0))],
        out_shape=[jax.ShapeDtypeStruct((SEG_PAD, D), jnp.float32)] * 2,
    )(acc, w4, w5)


def _comb_out_body(acc_ref, o_ref):
    o_ref[...] = acc_ref[0] + acc_ref[1]


def _comb_out(acc):
    return pl.pallas_call(
        _comb_out_body,
        grid=(SEG_PAD // _CB,),
        in_specs=[pl.BlockSpec((NC, _CB, D), lambda i: (0, i, 0))],
        out_specs=pl.BlockSpec((_CB, D), lambda i: (i, 0)),
        out_shape=jax.ShapeDtypeStruct((SEG_PAD, D), jnp.float32),
    )(acc)


def kernel(p, q, hedge_index, W1, W2, W3, W4, W5, W6):
    node_idx = hedge_index[0]
    hedge_idx = hedge_index[1]
    q_p, q_pp, p_ppp = _mm3(q, W1, W2, W6)
    q_ppp = _mm1(p, W3)
    e1, pmax1 = _edge_scores(True)(q_ppp, node_idx, q_pp, hedge_idx)
    m1 = _combine_max()(pmax1)
    w1, ps1 = _weights()(node_idx, e1, m1)
    rinv1 = _combine_sum_recip()(ps1)
    acc1 = _messages()(q_p, hedge_idx, node_idx, w1, rinv1)
    p_p, p_pp = _comb_mm(acc1, W4, W5)
    e2, pmax2 = _edge_scores(False)(p_pp, node_idx, p_ppp, hedge_idx)
    m2 = _combine_max()(pmax2)
    w2, ps2 = _weights()(hedge_idx, e2, m2)
    rinv2 = _combine_sum_recip()(ps2)
    acc2 = _messages()(p_p, node_idx, hedge_idx, w2, rinv2)
    return _comb_out(acc2)[:M]
